# Initial kernel scaffold; baseline (speedup 1.0000x reference)
import jax
import jax.numpy as jnp
from jax.experimental import pallas as pl
from jax.experimental.pallas import tpu as pltpu

Q = 1024
D = 64
N = 100000
B = 2048
NBLK = 49
NPAD = NBLK * B
K = 5
LANES = 128
CHUNKS = B // LANES
QS = 64
PAD_VAL = 1e4
BIG = 1e30


def _knn_body(x_ref, data_ref, labels_ref, out_ref, dist_ref, *regs):
    vrefs = regs[:K]
    lrefs = regs[K:]
    n = pl.program_id(0)

    @pl.when(n == 0)
    def _init():
        for k in range(K):
            vrefs[k][:] = jnp.full((Q, LANES), BIG, jnp.float32)
            lrefs[k][:] = jnp.zeros((Q, LANES), jnp.int32)

    x = x_ref[:]
    d = data_ref[:]
    lab = labels_ref[0]

    xsq = jnp.sum(x * x, axis=1, keepdims=True)
    dsq = jnp.sum(d * d, axis=1, keepdims=True)
    zx = jnp.zeros((Q, LANES - D - 1), jnp.float32)
    zd = jnp.zeros((B, LANES - D - 1), jnp.float32)
    xa = jnp.concatenate([x * (-2.0), jnp.ones((Q, 1), jnp.float32), zx], axis=1)
    da = jnp.concatenate([d, dsq, zd], axis=1)
    cross = jax.lax.dot_general(
        xa, da, (((1,), (1,)), ((), ())),
        precision=jax.lax.Precision.HIGHEST,
        preferred_element_type=jnp.float32)
    dist_ref[:] = xsq + cross

    for qb in range(Q // QS):
        qlo = qb * QS
        vals = [vrefs[k][qlo:qlo + QS, :] for k in range(K)]
        labs = [lrefs[k][qlo:qlo + QS, :] for k in range(K)]
        for c in range(CHUNKS):
            clo = c * LANES
            v = dist_ref[qlo:qlo + QS, clo:clo + LANES]
            l = jnp.broadcast_to(lab[:, clo:clo + LANES], (QS, LANES))
            for k in range(K):
                cond = v < vals[k]
                nv = jnp.minimum(v, vals[k])
                xv = jnp.maximum(v, vals[k])
                nl = jnp.where(cond, l, labs[k])
                xl = jnp.where(cond, labs[k], l)
                vals[k] = nv
                labs[k] = nl
                v = xv
                l = xl
        for k in range(K):
            vrefs[k][qlo:qlo + QS, :] = vals[k]
            lrefs[k][qlo:qlo + QS, :] = labs[k]

    @pl.when(n == NBLK - 1)
    def _finish():
        V = jnp.concatenate([vrefs[k][:] for k in range(K)], axis=1)
        L = jnp.concatenate([lrefs[k][:] for k in range(K)], axis=1)
        cols = jax.lax.broadcasted_iota(jnp.int32, (Q, K * LANES), 1)
        knn_labs = []
        for _ in range(K):
            m = jnp.min(V, axis=1, keepdims=True)
            pos = jnp.min(jnp.where(V == m, cols, jnp.int32(1 << 30)),
                          axis=1, keepdims=True)
            sel = cols == pos
            knn_labs.append(jnp.sum(jnp.where(sel, L, 0), axis=1, keepdims=True))
            V = jnp.where(sel, BIG, V)
        best = jnp.full((Q, 1), -1, jnp.int32)
        pred = jnp.zeros((Q, 1), jnp.int32)
        for i in range(K):
            cnt = knn_labs[0] * 0
            for j in range(K):
                cnt = cnt + (knn_labs[i] == knn_labs[j]).astype(jnp.int32)
            score = cnt * 16384 - knn_labs[i]
            take = score > best
            best = jnp.where(take, score, best)
            pred = jnp.where(take, knn_labs[i], pred)
        out_ref[:] = pred


def kernel(x, data, labels):
    pad = NPAD - N
    data_p = jnp.concatenate(
        [data, jnp.full((pad, D), PAD_VAL, data.dtype)], axis=0)
    labels_p = jnp.concatenate(
        [labels, jnp.zeros((pad,), labels.dtype)], axis=0).reshape(NBLK, 1, B)

    preds = pl.pallas_call(
        _knn_body,
        grid=(NBLK,),
        in_specs=[
            pl.BlockSpec((Q, D), lambda n: (0, 0)),
            pl.BlockSpec((B, D), lambda n: (n, 0)),
            pl.BlockSpec((1, 1, B), lambda n: (n, 0, 0)),
        ],
        out_specs=pl.BlockSpec((Q, 1), lambda n: (0, 0)),
        out_shape=jax.ShapeDtypeStruct((Q, 1), jnp.int32),
        scratch_shapes=(
            [pltpu.VMEM((Q, B), jnp.float32)]
            + [pltpu.VMEM((Q, LANES), jnp.float32) for _ in range(K)]
            + [pltpu.VMEM((Q, LANES), jnp.int32) for _ in range(K)]
        ),
        compiler_params=pltpu.CompilerParams(
            dimension_semantics=("arbitrary",)),
    )(x, data_p, labels_p)
    return preds.reshape(Q)


# speedup vs baseline: 3.7534x; 3.7534x over previous
import jax
import jax.numpy as jnp
from jax.experimental import pallas as pl
from jax.experimental.pallas import tpu as pltpu

Q = 1024
D = 64
N = 100000
B = 2048
NBLK = 49
NPAD = NBLK * B
K = 5
LANES = 128
CHUNKS = B // LANES
QS = 64
PAD_VAL = 1e4
BIG = 1e30


def _knn_body(x_ref, data_ref, dsq_ref, labels_ref, out_ref, dist_ref, *regs):
    vrefs = regs[:K]
    lrefs = regs[K:]
    n = pl.program_id(0)

    @pl.when(n == 0)
    def _init():
        for k in range(K):
            vrefs[k][:] = jnp.full((Q, LANES), BIG, jnp.float32)
            lrefs[k][:] = jnp.zeros((Q, LANES), jnp.int32)

    x = x_ref[:]
    d = data_ref[:]
    lab = labels_ref[0]
    dsq = dsq_ref[0]

    xsq = jnp.sum(x * x, axis=1, keepdims=True)
    cross = jax.lax.dot_general(
        x, d, (((1,), (1,)), ((), ())),
        precision=jax.lax.Precision.DEFAULT,
        preferred_element_type=jnp.float32)
    dist_ref[:] = (xsq + dsq) - 2.0 * cross

    for qb in range(Q // QS):
        qlo = qb * QS
        vals = [vrefs[k][qlo:qlo + QS, :] for k in range(K)]
        labs = [lrefs[k][qlo:qlo + QS, :] for k in range(K)]
        for c in range(CHUNKS):
            clo = c * LANES
            v = dist_ref[qlo:qlo + QS, clo:clo + LANES]
            l = jnp.broadcast_to(lab[:, clo:clo + LANES], (QS, LANES))
            for k in range(K):
                cond = v < vals[k]
                nv = jnp.minimum(v, vals[k])
                xv = jnp.maximum(v, vals[k])
                nl = jnp.where(cond, l, labs[k])
                xl = jnp.where(cond, labs[k], l)
                vals[k] = nv
                labs[k] = nl
                v = xv
                l = xl
        for k in range(K):
            vrefs[k][qlo:qlo + QS, :] = vals[k]
            lrefs[k][qlo:qlo + QS, :] = labs[k]

    @pl.when(n == NBLK - 1)
    def _finish():
        V = jnp.concatenate([vrefs[k][:] for k in range(K)], axis=1)
        L = jnp.concatenate([lrefs[k][:] for k in range(K)], axis=1)
        cols = jax.lax.broadcasted_iota(jnp.int32, (Q, K * LANES), 1)
        knn_labs = []
        for _ in range(K):
            m = jnp.min(V, axis=1, keepdims=True)
            pos = jnp.min(jnp.where(V == m, cols, jnp.int32(1 << 30)),
                          axis=1, keepdims=True)
            sel = cols == pos
            knn_labs.append(jnp.sum(jnp.where(sel, L, 0), axis=1, keepdims=True))
            V = jnp.where(sel, BIG, V)
        best = jnp.full((Q, 1), -1, jnp.int32)
        pred = jnp.zeros((Q, 1), jnp.int32)
        for i in range(K):
            cnt = knn_labs[0] * 0
            for j in range(K):
                cnt = cnt + (knn_labs[i] == knn_labs[j]).astype(jnp.int32)
            score = cnt * 16384 - knn_labs[i]
            take = score > best
            best = jnp.where(take, score, best)
            pred = jnp.where(take, knn_labs[i], pred)
        out_ref[:] = pred


def kernel(x, data, labels):
    pad = NPAD - N
    data_p = jnp.concatenate(
        [data, jnp.zeros((pad, D), data.dtype)], axis=0)
    dsq = jnp.sum(data * data, axis=1)
    dsq_p = jnp.concatenate(
        [dsq, jnp.full((pad,), 1e10, jnp.float32)], axis=0).reshape(NBLK, 1, B)
    labels_p = jnp.concatenate(
        [labels, jnp.zeros((pad,), labels.dtype)], axis=0).reshape(NBLK, 1, B)

    preds = pl.pallas_call(
        _knn_body,
        grid=(NBLK,),
        in_specs=[
            pl.BlockSpec((Q, D), lambda n: (0, 0)),
            pl.BlockSpec((B, D), lambda n: (n, 0)),
            pl.BlockSpec((1, 1, B), lambda n: (n, 0, 0)),
            pl.BlockSpec((1, 1, B), lambda n: (n, 0, 0)),
        ],
        out_specs=pl.BlockSpec((Q, 1), lambda n: (0, 0)),
        out_shape=jax.ShapeDtypeStruct((Q, 1), jnp.int32),
        scratch_shapes=(
            [pltpu.VMEM((Q, B), jnp.float32)]
            + [pltpu.VMEM((Q, LANES), jnp.float32) for _ in range(K)]
            + [pltpu.VMEM((Q, LANES), jnp.int32) for _ in range(K)]
        ),
        compiler_params=pltpu.CompilerParams(
            dimension_semantics=("arbitrary",)),
    )(x, data_p, dsq_p, labels_p)
    return preds.reshape(Q)
